# initial kernel scaffold (unmeasured)
import functools

import jax
import jax.numpy as jnp
from jax import lax
from jax.experimental import pallas as pl
from jax.experimental.pallas import tpu as pltpu

B, S, HD_SHARD, N = 4, 1024, 2048, 4096
S_HALF = S // 2

M_TILE = 512
N_TILE = 1024


def _matmul_body(a_ref, w_ref, out_ref):
    out_ref[:, :] = jnp.dot(
        a_ref[:, :], w_ref[:, :], preferred_element_type=jnp.float32
    )


def _matmul(a, w):
    m, k = a.shape
    _, n = w.shape
    return pl.pallas_call(
        _matmul_body,
        grid=(m // M_TILE, n // N_TILE),
        in_specs=[
            pl.BlockSpec((M_TILE, k), lambda i, j: (i, 0)),
            pl.BlockSpec((k, N_TILE), lambda i, j: (0, j)),
        ],
        out_specs=pl.BlockSpec((M_TILE, N_TILE), lambda i, j: (i, j)),
        out_shape=jax.ShapeDtypeStruct((m, n), jnp.float32),
    )(a, w)


def _exchange_body(p_ref, recv_ref, loc_ref, send_sem, recv_sem, copy_sem):
    my_x = lax.axis_index("x")
    my_y = lax.axis_index("y")

    barrier_sem = pltpu.get_barrier_semaphore()
    pl.semaphore_signal(
        barrier_sem,
        inc=1,
        device_id=(my_x, 1 - my_y),
        device_id_type=pl.DeviceIdType.MESH,
    )
    pl.semaphore_wait(barrier_sem, 1)

    rdma = pltpu.make_async_remote_copy(
        src_ref=p_ref.at[:, 1 - my_y],
        dst_ref=recv_ref,
        send_sem=send_sem,
        recv_sem=recv_sem,
        device_id=(my_x, 1 - my_y),
        device_id_type=pl.DeviceIdType.MESH,
    )
    rdma.start()

    local = pltpu.make_async_copy(p_ref.at[:, my_y], loc_ref, copy_sem)
    local.start()
    local.wait()
    rdma.wait()


def _exchange(p4):
    half = jax.ShapeDtypeStruct((B, S_HALF, N), jnp.float32)
    return pl.pallas_call(
        _exchange_body,
        in_specs=[pl.BlockSpec(memory_space=pltpu.ANY)],
        out_specs=(
            pl.BlockSpec(memory_space=pltpu.ANY),
            pl.BlockSpec(memory_space=pltpu.ANY),
        ),
        out_shape=(half, half),
        scratch_shapes=[
            pltpu.SemaphoreType.DMA,
            pltpu.SemaphoreType.DMA,
            pltpu.SemaphoreType.DMA,
        ],
        compiler_params=pltpu.CompilerParams(collective_id=0),
    )(p4)


def _add_body(a_ref, b_ref, out_ref):
    out_ref[...] = a_ref[...] + b_ref[...]


def _add(a, b):
    return pl.pallas_call(
        _add_body,
        grid=(B, N // N_TILE),
        in_specs=[
            pl.BlockSpec((1, S_HALF, N_TILE), lambda i, j: (i, 0, j)),
            pl.BlockSpec((1, S_HALF, N_TILE), lambda i, j: (i, 0, j)),
        ],
        out_specs=pl.BlockSpec((1, S_HALF, N_TILE), lambda i, j: (i, 0, j)),
        out_shape=jax.ShapeDtypeStruct((B, S_HALF, N), jnp.float32),
    )(a, b)


def kernel(O, Wo):
    a = O.reshape(B * S, HD_SHARD)
    partial = _matmul(a, Wo)
    p4 = partial.reshape(B, 2, S_HALF, N)
    recv, loc = _exchange(p4)
    return _add(recv, loc)


# baseline (device time: 1211604 ns/iter reference)
import functools

import jax
import jax.numpy as jnp
from jax import lax
from jax.experimental import pallas as pl
from jax.experimental.pallas import tpu as pltpu

B, S, HD_SHARD, N = 4, 1024, 2048, 4096
S_HALF = S // 2

M_TILE = 512
N_TILE = 1024


def _matmul_body(a_ref, w_ref, out_ref):
    out_ref[:, :] = jnp.dot(
        a_ref[:, :], w_ref[:, :], preferred_element_type=jnp.float32
    )


def _matmul(a, w):
    m, k = a.shape
    _, n = w.shape
    return pl.pallas_call(
        _matmul_body,
        grid=(m // M_TILE, n // N_TILE),
        in_specs=[
            pl.BlockSpec((M_TILE, k), lambda i, j: (i, 0)),
            pl.BlockSpec((k, N_TILE), lambda i, j: (0, j)),
        ],
        out_specs=pl.BlockSpec((M_TILE, N_TILE), lambda i, j: (i, j)),
        out_shape=jax.ShapeDtypeStruct((m, n), jnp.float32),
    )(a, w)


def _exchange_body(p_ref, recv_ref, loc_ref, send_sem, recv_sem, copy_sem):
    my_x = lax.axis_index("x")
    my_y = lax.axis_index("y")

    barrier_sem = pltpu.get_barrier_semaphore()
    pl.semaphore_signal(
        barrier_sem,
        inc=1,
        device_id=(my_x, 1 - my_y),
        device_id_type=pl.DeviceIdType.MESH,
    )
    pl.semaphore_wait(barrier_sem, 1)

    rdma = pltpu.make_async_remote_copy(
        src_ref=p_ref.at[:, 1 - my_y],
        dst_ref=recv_ref,
        send_sem=send_sem,
        recv_sem=recv_sem,
        device_id=(my_x, 1 - my_y),
        device_id_type=pl.DeviceIdType.MESH,
    )
    rdma.start()

    local = pltpu.make_async_copy(p_ref.at[:, my_y], loc_ref, copy_sem)
    local.start()
    local.wait()
    rdma.wait()


def _exchange(p4):
    half = jax.ShapeDtypeStruct((B, S_HALF, N), jnp.float32)
    return pl.pallas_call(
        _exchange_body,
        in_specs=[pl.BlockSpec(memory_space=pl.ANY)],
        out_specs=(
            pl.BlockSpec(memory_space=pl.ANY),
            pl.BlockSpec(memory_space=pl.ANY),
        ),
        out_shape=(half, half),
        scratch_shapes=[
            pltpu.SemaphoreType.DMA,
            pltpu.SemaphoreType.DMA,
            pltpu.SemaphoreType.DMA,
        ],
        compiler_params=pltpu.CompilerParams(collective_id=0),
    )(p4)


def _add_body(a_ref, b_ref, out_ref):
    out_ref[...] = a_ref[...] + b_ref[...]


def _add(a, b):
    return pl.pallas_call(
        _add_body,
        grid=(B, N // N_TILE),
        in_specs=[
            pl.BlockSpec((1, S_HALF, N_TILE), lambda i, j: (i, 0, j)),
            pl.BlockSpec((1, S_HALF, N_TILE), lambda i, j: (i, 0, j)),
        ],
        out_specs=pl.BlockSpec((1, S_HALF, N_TILE), lambda i, j: (i, 0, j)),
        out_shape=jax.ShapeDtypeStruct((B, S_HALF, N), jnp.float32),
    )(a, b)


def kernel(O, Wo):
    a = O.reshape(B * S, HD_SHARD)
    partial = _matmul(a, Wo)
    p4 = partial.reshape(B, 2, S_HALF, N)
    recv, loc = _exchange(p4)
    return _add(recv, loc)


# device time: 551739 ns/iter; 2.1960x vs baseline; 2.1960x over previous
import jax
import jax.numpy as jnp
from jax import lax
from jax.experimental import pallas as pl
from jax.experimental.pallas import tpu as pltpu

B, S, HD_SHARD, N = 4, 1024, 2048, 4096
S_HALF = S // 2
M_TILE = 512
N_TILE = 1024
HALF_ROWS = B * S_HALF


def _matmul_body(a_ref, w_ref, out_ref):
    out_ref[:, :] = jnp.dot(
        a_ref[:, :], w_ref[:, :], preferred_element_type=jnp.float32
    )


def _a_index_map(g, n):
    my_y = lax.axis_index("y")
    h = jnp.where(g < 4, my_y, 1 - my_y)
    return ((g % 4) * 2 + h, 0)


def _matmul(a, w):
    m, k = a.shape
    _, n = w.shape
    return pl.pallas_call(
        _matmul_body,
        grid=(m // M_TILE, n // N_TILE),
        in_specs=[
            pl.BlockSpec((M_TILE, k), _a_index_map),
            pl.BlockSpec((k, N_TILE), lambda g, n_: (0, n_)),
        ],
        out_specs=pl.BlockSpec((M_TILE, N_TILE), lambda g, n_: (g, n_)),
        out_shape=jax.ShapeDtypeStruct((m, n), jnp.float32),
    )(a, w)


def _exchange_body(p_ref, recv_ref, send_sem, recv_sem):
    my_x = lax.axis_index("x")
    my_y = lax.axis_index("y")

    barrier_sem = pltpu.get_barrier_semaphore()
    pl.semaphore_signal(
        barrier_sem,
        inc=1,
        device_id=(my_x, 1 - my_y),
        device_id_type=pl.DeviceIdType.MESH,
    )
    pl.semaphore_wait(barrier_sem, 1)

    rdma = pltpu.make_async_remote_copy(
        src_ref=p_ref.at[pl.ds(HALF_ROWS, HALF_ROWS)],
        dst_ref=recv_ref,
        send_sem=send_sem,
        recv_sem=recv_sem,
        device_id=(my_x, 1 - my_y),
        device_id_type=pl.DeviceIdType.MESH,
    )
    rdma.start()
    rdma.wait()


def _exchange(p):
    return pl.pallas_call(
        _exchange_body,
        in_specs=[pl.BlockSpec(memory_space=pl.ANY)],
        out_specs=pl.BlockSpec(memory_space=pl.ANY),
        out_shape=jax.ShapeDtypeStruct((HALF_ROWS, N), jnp.float32),
        scratch_shapes=[pltpu.SemaphoreType.DMA, pltpu.SemaphoreType.DMA],
        compiler_params=pltpu.CompilerParams(collective_id=0),
    )(p)


def _add_body(p_ref, r_ref, out_ref):
    out_ref[0] = p_ref[...] + r_ref[...]


def _add(p, recv):
    return pl.pallas_call(
        _add_body,
        grid=(B, N // N_TILE),
        in_specs=[
            pl.BlockSpec((S_HALF, N_TILE), lambda b, n_: (b, n_)),
            pl.BlockSpec((S_HALF, N_TILE), lambda b, n_: (b, n_)),
        ],
        out_specs=pl.BlockSpec((1, S_HALF, N_TILE), lambda b, n_: (b, 0, n_)),
        out_shape=jax.ShapeDtypeStruct((B, S_HALF, N), jnp.float32),
    )(p, recv)


def kernel(O, Wo):
    a = O.reshape(B * S, HD_SHARD)
    partial = _matmul(a, Wo)
    recv = _exchange(partial)
    return _add(partial, recv)


# device time: 446265 ns/iter; 2.7150x vs baseline; 1.2363x over previous
import jax
import jax.numpy as jnp
from jax import lax
from jax.experimental import pallas as pl
from jax.experimental.pallas import tpu as pltpu

B, S, HD_SHARD, N = 4, 1024, 2048, 4096
S_HALF = S // 2
M_TILE = 512
N_TILE = 1024
N_STEPS = N // N_TILE
HALF_ROWS = B * S_HALF


def _a_index_map(g, n):
    my_y = lax.axis_index("y")
    h = jnp.where(g < 4, 1 - my_y, my_y)
    return ((g % 4) * 2 + h, 0)


def _mm_body(a_ref, w_ref, out_ref, recv_ref, send_buf, send_sems, recv_sems):
    g = pl.program_id(0)
    n = pl.program_id(1)
    my_x = lax.axis_index("x")
    my_y = lax.axis_index("y")
    peer = (my_x, 1 - my_y)

    def tile_rdma(tile, slot):
        return pltpu.make_async_remote_copy(
            src_ref=send_buf.at[slot],
            dst_ref=recv_ref.at[pl.ds(tile * M_TILE, M_TILE)],
            send_sem=send_sems.at[slot],
            recv_sem=recv_sems.at[tile],
            device_id=peer,
            device_id_type=pl.DeviceIdType.MESH,
        )

    @pl.when((g == 0) & (n == 0))
    def _():
        barrier_sem = pltpu.get_barrier_semaphore()
        pl.semaphore_signal(
            barrier_sem,
            inc=1,
            device_id=peer,
            device_id_type=pl.DeviceIdType.MESH,
        )
        pl.semaphore_wait(barrier_sem, 1)

    @pl.when((g >= 2) & (g < 4) & (n == 0))
    def _():
        tile_rdma(g - 2, g % 2).wait_send()

    acc = jnp.dot(a_ref[:, :], w_ref[:, :], preferred_element_type=jnp.float32)
    out_ref[:, :] = acc

    @pl.when(g < 4)
    def _():
        send_buf[g % 2, :, pl.ds(n * N_TILE, N_TILE)] = acc

    @pl.when((g < 4) & (n == N_STEPS - 1))
    def _():
        tile_rdma(g, g % 2).start()

    @pl.when((g == 7) & (n == N_STEPS - 1))
    def _():
        tile_rdma(2, 0).wait_send()
        tile_rdma(3, 1).wait_send()
        for t in range(4):
            tile_rdma(t, t % 2).wait_recv()


def _mm_send(a, w):
    return pl.pallas_call(
        _mm_body,
        grid=(2 * B, N_STEPS),
        in_specs=[
            pl.BlockSpec((M_TILE, HD_SHARD), _a_index_map),
            pl.BlockSpec((HD_SHARD, N_TILE), lambda g, n: (0, n)),
        ],
        out_specs=(
            pl.BlockSpec((M_TILE, N_TILE), lambda g, n: (g % 4, n)),
            pl.BlockSpec(memory_space=pl.ANY),
        ),
        out_shape=(
            jax.ShapeDtypeStruct((HALF_ROWS, N), jnp.float32),
            jax.ShapeDtypeStruct((HALF_ROWS, N), jnp.float32),
        ),
        scratch_shapes=[
            pltpu.VMEM((2, M_TILE, N), jnp.float32),
            pltpu.SemaphoreType.DMA((2,)),
            pltpu.SemaphoreType.DMA((4,)),
        ],
        compiler_params=pltpu.CompilerParams(
            collective_id=0,
            vmem_limit_bytes=56 * 1024 * 1024,
        ),
    )(a, w)


def _add_body(p_ref, r_ref, out_ref):
    out_ref[0] = p_ref[...] + r_ref[...]


def _add(p, recv):
    return pl.pallas_call(
        _add_body,
        grid=(B, N // N_TILE),
        in_specs=[
            pl.BlockSpec((S_HALF, N_TILE), lambda b, n_: (b, n_)),
            pl.BlockSpec((S_HALF, N_TILE), lambda b, n_: (b, n_)),
        ],
        out_specs=pl.BlockSpec((1, S_HALF, N_TILE), lambda b, n_: (b, 0, n_)),
        out_shape=jax.ShapeDtypeStruct((B, S_HALF, N), jnp.float32),
    )(p, recv)


def kernel(O, Wo):
    a = O.reshape(B * S, HD_SHARD)
    mine, recv = _mm_send(a, Wo)
    return _add(mine, recv)


# device time: 422648 ns/iter; 2.8667x vs baseline; 1.0559x over previous
import jax
import jax.numpy as jnp
from jax import lax
from jax.experimental import pallas as pl
from jax.experimental.pallas import tpu as pltpu

B, S, HD_SHARD, N = 4, 1024, 2048, 4096
H, D = 16, 128
S_HALF = S // 2
M_TILE = 512
N_TILE = 1024
N_STEPS = N // N_TILE
HALF_ROWS = B * S_HALF


def _a_index_map(g, n):
    my_y = lax.axis_index("y")
    h = jnp.where(g < 4, 1 - my_y, my_y)
    return (g % 4, h, 0, 0)


def _mm_body(a_ref, w_ref, out_ref, recv_ref, send_buf, send_sems, recv_sems):
    g = pl.program_id(0)
    n = pl.program_id(1)
    my_x = lax.axis_index("x")
    my_y = lax.axis_index("y")
    peer = (my_x, 1 - my_y)

    def tile_rdma(tile, slot):
        return pltpu.make_async_remote_copy(
            src_ref=send_buf.at[slot],
            dst_ref=recv_ref.at[pl.ds(tile * M_TILE, M_TILE)],
            send_sem=send_sems.at[slot],
            recv_sem=recv_sems.at[tile],
            device_id=peer,
            device_id_type=pl.DeviceIdType.MESH,
        )

    @pl.when((g == 0) & (n == 0))
    def _():
        barrier_sem = pltpu.get_barrier_semaphore()
        pl.semaphore_signal(
            barrier_sem,
            inc=1,
            device_id=peer,
            device_id_type=pl.DeviceIdType.MESH,
        )
        pl.semaphore_wait(barrier_sem, 1)

    @pl.when((g >= 2) & (g < 4) & (n == 0))
    def _():
        tile_rdma(g - 2, g % 2).wait_send()

    acc = jnp.dot(
        a_ref[0, :, 0, :],
        w_ref[0:D, :],
        preferred_element_type=jnp.float32,
    )
    for h in range(1, H):
        acc += jnp.dot(
            a_ref[0, :, h, :],
            w_ref[h * D : (h + 1) * D, :],
            preferred_element_type=jnp.float32,
        )
    out_ref[:, :] = acc

    @pl.when(g < 4)
    def _():
        send_buf[g % 2, :, pl.ds(n * N_TILE, N_TILE)] = acc

    @pl.when((g < 4) & (n == N_STEPS - 1))
    def _():
        tile_rdma(g, g % 2).start()

    @pl.when((g == 7) & (n == N_STEPS - 1))
    def _():
        tile_rdma(2, 0).wait_send()
        tile_rdma(3, 1).wait_send()
        for t in range(4):
            tile_rdma(t, t % 2).wait_recv()


def _mm_send(a, w):
    return pl.pallas_call(
        _mm_body,
        grid=(2 * B, N_STEPS),
        in_specs=[
            pl.BlockSpec((1, M_TILE, H, D), _a_index_map),
            pl.BlockSpec((HD_SHARD, N_TILE), lambda g, n: (0, n)),
        ],
        out_specs=(
            pl.BlockSpec((M_TILE, N_TILE), lambda g, n: (g % 4, n)),
            pl.BlockSpec(memory_space=pl.ANY),
        ),
        out_shape=(
            jax.ShapeDtypeStruct((HALF_ROWS, N), jnp.float32),
            jax.ShapeDtypeStruct((HALF_ROWS, N), jnp.float32),
        ),
        scratch_shapes=[
            pltpu.VMEM((2, M_TILE, N), jnp.float32),
            pltpu.SemaphoreType.DMA((2,)),
            pltpu.SemaphoreType.DMA((4,)),
        ],
        compiler_params=pltpu.CompilerParams(
            collective_id=0,
            vmem_limit_bytes=56 * 1024 * 1024,
        ),
    )(a, w)


def _add_body(p_ref, r_ref, out_ref):
    out_ref[0] = p_ref[...] + r_ref[...]


def _add(p, recv):
    return pl.pallas_call(
        _add_body,
        grid=(B, N // N_TILE),
        in_specs=[
            pl.BlockSpec((S_HALF, N_TILE), lambda b, n_: (b, n_)),
            pl.BlockSpec((S_HALF, N_TILE), lambda b, n_: (b, n_)),
        ],
        out_specs=pl.BlockSpec((1, S_HALF, N_TILE), lambda b, n_: (b, 0, n_)),
        out_shape=jax.ShapeDtypeStruct((B, S_HALF, N), jnp.float32),
    )(p, recv)


def kernel(O, Wo):
    mine, recv = _mm_send(O, Wo)
    return _add(mine, recv)


# device time: 399587 ns/iter; 3.0321x vs baseline; 1.0577x over previous
import jax
import jax.numpy as jnp
from jax import lax
from jax.experimental import pallas as pl
from jax.experimental.pallas import tpu as pltpu

B, S, HD_SHARD, N = 4, 1024, 2048, 4096
H, D = 16, 128
S_HALF = S // 2
M_TILE = 512
N_TILE = 1024
N_STEPS = N // N_TILE
N_CHUNKS = B * N_STEPS


def _a_index_map(g, n):
    my_y = lax.axis_index("y")
    h = jnp.where(g < 4, 1 - my_y, my_y)
    return (g % 4, h, 0, 0)


def _body(a_ref, w_ref, out_ref, recv_ref, send_buf, add_buf,
          load_sem, send_sems, recv_sems):
    g = pl.program_id(0)
    n = pl.program_id(1)
    my_x = lax.axis_index("x")
    my_y = lax.axis_index("y")
    peer = (my_x, 1 - my_y)

    def chunk_rdma(tile, cn):
        return pltpu.make_async_remote_copy(
            src_ref=send_buf.at[tile % 2, :, pl.ds(cn * N_TILE, N_TILE)],
            dst_ref=recv_ref.at[tile * N_STEPS + cn],
            send_sem=send_sems.at[tile % 2, cn],
            recv_sem=recv_sems.at[tile * N_STEPS + cn],
            device_id=peer,
            device_id_type=pl.DeviceIdType.MESH,
        )

    @pl.when((g == 0) & (n == 0))
    def _():
        barrier_sem = pltpu.get_barrier_semaphore()
        pl.semaphore_signal(
            barrier_sem,
            inc=1,
            device_id=peer,
            device_id_type=pl.DeviceIdType.MESH,
        )
        pl.semaphore_wait(barrier_sem, 1)

    @pl.when((g >= 2) & (g < 4) & (n == 0))
    def _():
        for cn in range(N_STEPS):
            chunk_rdma(g - 2, cn).wait_send()

    acc = jnp.dot(
        a_ref[0, :, 0, :], w_ref[0:D, :], preferred_element_type=jnp.float32
    )
    for h in range(1, H):
        acc += jnp.dot(
            a_ref[0, :, h, :],
            w_ref[h * D : (h + 1) * D, :],
            preferred_element_type=jnp.float32,
        )

    @pl.when(g < 4)
    def _():
        out_ref[0] = acc
        send_buf[g % 2, :, pl.ds(n * N_TILE, N_TILE)] = acc
        chunk_rdma(g, n).start()

    @pl.when(g >= 4)
    def _():
        chunk_rdma(g - 4, n).wait_recv()
        load = pltpu.make_async_copy(
            recv_ref.at[(g - 4) * N_STEPS + n], add_buf, load_sem
        )
        load.start()
        load.wait()
        out_ref[0] = acc + add_buf[...]

    @pl.when((g == 7) & (n == N_STEPS - 1))
    def _():
        for t in (2, 3):
            for cn in range(N_STEPS):
                chunk_rdma(t, cn).wait_send()


def kernel(O, Wo):
    out, _recv = pl.pallas_call(
        _body,
        grid=(2 * B, N_STEPS),
        in_specs=[
            pl.BlockSpec((1, M_TILE, H, D), _a_index_map),
            pl.BlockSpec((HD_SHARD, N_TILE), lambda g, n: (0, n)),
        ],
        out_specs=(
            pl.BlockSpec((1, M_TILE, N_TILE), lambda g, n: (g % 4, 0, n)),
            pl.BlockSpec(memory_space=pl.ANY),
        ),
        out_shape=(
            jax.ShapeDtypeStruct((B, S_HALF, N), jnp.float32),
            jax.ShapeDtypeStruct((N_CHUNKS, M_TILE, N_TILE), jnp.float32),
        ),
        scratch_shapes=[
            pltpu.VMEM((2, M_TILE, N), jnp.float32),
            pltpu.VMEM((M_TILE, N_TILE), jnp.float32),
            pltpu.SemaphoreType.DMA,
            pltpu.SemaphoreType.DMA((2, N_STEPS)),
            pltpu.SemaphoreType.DMA((N_CHUNKS,)),
        ],
        compiler_params=pltpu.CompilerParams(
            collective_id=0,
            vmem_limit_bytes=56 * 1024 * 1024,
        ),
    )(O, Wo)
    return out


# device time: 323812 ns/iter; 3.7417x vs baseline; 1.2340x over previous
import jax
import jax.numpy as jnp
from jax import lax
from jax.experimental import pallas as pl
from jax.experimental.pallas import tpu as pltpu

B, S, HD_SHARD, N = 4, 1024, 2048, 4096
H, D = 16, 128
S_HALF = S // 2
M_TILE = 512
N_TILE = 1024
N_SUB = 2
N_CHUNKS = B * N_SUB


def _a_index_map(g, n):
    my_y = lax.axis_index("y")
    h = jnp.where(g < 4, 1 - my_y, my_y)
    return (g % 4, h, 0, 0)


def _w_index_map(g, n):
    my_x = lax.axis_index("x")
    return (0, my_x * N_SUB + n)


def _body(a_ref, w_ref, out_ref, land_ref, send_buf, out_slots, add_buf,
          load_sem, store_sems, ysend_sems, yrecv_sems, xsend_sems,
          xrecv_sems):
    g = pl.program_id(0)
    n = pl.program_id(1)
    my_x = lax.axis_index("x")
    my_y = lax.axis_index("y")
    ypeer = (my_x, 1 - my_y)
    xpeer = (1 - my_x, my_y)
    col0 = my_x * (N_SUB * N_TILE)

    def y_rdma(tile, cn):
        return pltpu.make_async_remote_copy(
            src_ref=send_buf.at[tile % 2, :, pl.ds(cn * N_TILE, N_TILE)],
            dst_ref=land_ref.at[tile * N_SUB + cn],
            send_sem=ysend_sems.at[tile % 2, cn],
            recv_sem=yrecv_sems.at[tile * N_SUB + cn],
            device_id=ypeer,
            device_id_type=pl.DeviceIdType.MESH,
        )

    def x_rdma(c, tile, cn):
        return pltpu.make_async_remote_copy(
            src_ref=out_slots.at[c % 2],
            dst_ref=out_ref.at[tile, :, pl.ds(col0 + cn * N_TILE, N_TILE)],
            send_sem=xsend_sems.at[c % 2],
            recv_sem=xrecv_sems.at[c],
            device_id=xpeer,
            device_id_type=pl.DeviceIdType.MESH,
        )

    def out_store(c, tile, cn):
        return pltpu.make_async_copy(
            out_slots.at[c % 2],
            out_ref.at[tile, :, pl.ds(col0 + cn * N_TILE, N_TILE)],
            store_sems.at[c % 2],
        )

    @pl.when((g == 0) & (n == 0))
    def _():
        barrier_sem = pltpu.get_barrier_semaphore()
        for nbr in (ypeer, xpeer):
            pl.semaphore_signal(
                barrier_sem,
                inc=1,
                device_id=nbr,
                device_id_type=pl.DeviceIdType.MESH,
            )
        pl.semaphore_wait(barrier_sem, 2)

    @pl.when((g >= 2) & (g < 4) & (n == 0))
    def _():
        for cn in range(N_SUB):
            y_rdma(g - 2, cn).wait_send()

    acc = jnp.dot(
        a_ref[0, :, 0, :], w_ref[0:D, :], preferred_element_type=jnp.float32
    )
    for h in range(1, H):
        acc += jnp.dot(
            a_ref[0, :, h, :],
            w_ref[h * D : (h + 1) * D, :],
            preferred_element_type=jnp.float32,
        )

    @pl.when(g < 4)
    def _():
        send_buf[g % 2, :, pl.ds(n * N_TILE, N_TILE)] = acc
        y_rdma(g, n).start()

    @pl.when(g >= 4)
    def _():
        tile = g - 4
        c = tile * N_SUB + n
        @pl.when(c >= 2)
        def _():
            out_store(c - 2, 0, 0).wait()
            x_rdma(c - 2, 0, 0).wait_send()

        y_rdma(tile, n).wait_recv()
        load = pltpu.make_async_copy(land_ref.at[c], add_buf, load_sem)
        load.start()
        load.wait()
        out_slots[c % 2] = acc + add_buf[...]
        out_store(c, tile, n).start()
        x_rdma(c, tile, n).start()

    @pl.when((g == 7) & (n == N_SUB - 1))
    def _():
        for t in (2, 3):
            for cn in range(N_SUB):
                y_rdma(t, cn).wait_send()
        for c in (N_CHUNKS - 2, N_CHUNKS - 1):
            out_store(c, 0, 0).wait()
            x_rdma(c, 0, 0).wait_send()
        for c in range(N_CHUNKS):
            x_rdma(c, 0, 0).wait_recv()


def kernel(O, Wo):
    out, _land = pl.pallas_call(
        _body,
        grid=(2 * B, N_SUB),
        in_specs=[
            pl.BlockSpec((1, M_TILE, H, D), _a_index_map),
            pl.BlockSpec((HD_SHARD, N_TILE), _w_index_map),
        ],
        out_specs=(
            pl.BlockSpec(memory_space=pl.ANY),
            pl.BlockSpec(memory_space=pl.ANY),
        ),
        out_shape=(
            jax.ShapeDtypeStruct((B, S_HALF, N), jnp.float32),
            jax.ShapeDtypeStruct((N_CHUNKS, M_TILE, N_TILE), jnp.float32),
        ),
        scratch_shapes=[
            pltpu.VMEM((2, M_TILE, N_SUB * N_TILE), jnp.float32),
            pltpu.VMEM((2, M_TILE, N_TILE), jnp.float32),
            pltpu.VMEM((M_TILE, N_TILE), jnp.float32),
            pltpu.SemaphoreType.DMA,
            pltpu.SemaphoreType.DMA((2,)),
            pltpu.SemaphoreType.DMA((2, N_SUB)),
            pltpu.SemaphoreType.DMA((N_CHUNKS,)),
            pltpu.SemaphoreType.DMA((2,)),
            pltpu.SemaphoreType.DMA((N_CHUNKS,)),
        ],
        compiler_params=pltpu.CompilerParams(
            collective_id=0,
            vmem_limit_bytes=56 * 1024 * 1024,
        ),
    )(O, Wo)
    return out


# device time: 270580 ns/iter; 4.4778x vs baseline; 1.1967x over previous
import jax
import jax.numpy as jnp
from jax import lax
from jax.experimental import pallas as pl
from jax.experimental.pallas import tpu as pltpu

B, S, HD_SHARD, N = 4, 1024, 2048, 4096
H, D = 16, 128
S_HALF = S // 2
M_TILE = 512
N_TILE = 1024
N_SUB = 2
N_CHUNKS = B * N_SUB


def _a_index_map(g, n):
    my_y = lax.axis_index("y")
    h = jnp.where(g % 2 == 0, 1 - my_y, my_y)
    return (g // 2, h, 0, 0)


def _w_index_map(g, n):
    my_x = lax.axis_index("x")
    return (0, my_x * N_SUB + n)


def _body(a_ref, w_ref, out_ref, land_ref, send_buf, out_slots, add_buf,
          load_sem, store_sems, ysend_sems, yrecv_sems, xsend_sems,
          xrecv_sems):
    g = pl.program_id(0)
    n = pl.program_id(1)
    my_x = lax.axis_index("x")
    my_y = lax.axis_index("y")
    ypeer = (my_x, 1 - my_y)
    xpeer = (1 - my_x, my_y)
    col0 = my_x * (N_SUB * N_TILE)

    def y_rdma(tile, cn):
        return pltpu.make_async_remote_copy(
            src_ref=send_buf.at[tile % 2, :, pl.ds(cn * N_TILE, N_TILE)],
            dst_ref=land_ref.at[tile * N_SUB + cn],
            send_sem=ysend_sems.at[tile % 2, cn],
            recv_sem=yrecv_sems.at[tile * N_SUB + cn],
            device_id=ypeer,
            device_id_type=pl.DeviceIdType.MESH,
        )

    def x_rdma(c, tile, cn):
        return pltpu.make_async_remote_copy(
            src_ref=out_slots.at[c % 2],
            dst_ref=out_ref.at[tile, :, pl.ds(col0 + cn * N_TILE, N_TILE)],
            send_sem=xsend_sems.at[c % 2],
            recv_sem=xrecv_sems.at[c],
            device_id=xpeer,
            device_id_type=pl.DeviceIdType.MESH,
        )

    def out_store(c, tile, cn):
        return pltpu.make_async_copy(
            out_slots.at[c % 2],
            out_ref.at[tile, :, pl.ds(col0 + cn * N_TILE, N_TILE)],
            store_sems.at[c % 2],
        )

    tile = g // 2

    @pl.when((g == 0) & (n == 0))
    def _():
        barrier_sem = pltpu.get_barrier_semaphore()
        for nbr in (ypeer, xpeer):
            pl.semaphore_signal(
                barrier_sem,
                inc=1,
                device_id=nbr,
                device_id_type=pl.DeviceIdType.MESH,
            )
        pl.semaphore_wait(barrier_sem, 2)

    @pl.when((g >= 4) & (g % 2 == 0) & (n == 0))
    def _():
        for cn in range(N_SUB):
            y_rdma(tile - 2, cn).wait_send()

    acc = jnp.dot(
        a_ref[0, :, 0, :], w_ref[0:D, :], preferred_element_type=jnp.float32
    )
    for h in range(1, H):
        acc += jnp.dot(
            a_ref[0, :, h, :],
            w_ref[h * D : (h + 1) * D, :],
            preferred_element_type=jnp.float32,
        )

    @pl.when(g % 2 == 0)
    def _():
        send_buf[tile % 2, :, pl.ds(n * N_TILE, N_TILE)] = acc
        y_rdma(tile, n).start()

    @pl.when(g % 2 == 1)
    def _():
        c = tile * N_SUB + n
        @pl.when(c >= 2)
        def _():
            out_store(c - 2, 0, 0).wait()
            x_rdma(c - 2, 0, 0).wait_send()

        y_rdma(tile, n).wait_recv()
        load = pltpu.make_async_copy(land_ref.at[c], add_buf, load_sem)
        load.start()
        load.wait()
        out_slots[c % 2] = acc + add_buf[...]
        out_store(c, tile, n).start()
        x_rdma(c, tile, n).start()

    @pl.when((g == 7) & (n == N_SUB - 1))
    def _():
        for t in (2, 3):
            for cn in range(N_SUB):
                y_rdma(t, cn).wait_send()
        for c in (N_CHUNKS - 2, N_CHUNKS - 1):
            out_store(c, 0, 0).wait()
            x_rdma(c, 0, 0).wait_send()
        for c in range(N_CHUNKS):
            x_rdma(c, 0, 0).wait_recv()


def kernel(O, Wo):
    out, _land = pl.pallas_call(
        _body,
        grid=(2 * B, N_SUB),
        in_specs=[
            pl.BlockSpec((1, M_TILE, H, D), _a_index_map),
            pl.BlockSpec((HD_SHARD, N_TILE), _w_index_map),
        ],
        out_specs=(
            pl.BlockSpec(memory_space=pl.ANY),
            pl.BlockSpec(memory_space=pl.ANY),
        ),
        out_shape=(
            jax.ShapeDtypeStruct((B, S_HALF, N), jnp.float32),
            jax.ShapeDtypeStruct((N_CHUNKS, M_TILE, N_TILE), jnp.float32),
        ),
        scratch_shapes=[
            pltpu.VMEM((2, M_TILE, N_SUB * N_TILE), jnp.float32),
            pltpu.VMEM((2, M_TILE, N_TILE), jnp.float32),
            pltpu.VMEM((M_TILE, N_TILE), jnp.float32),
            pltpu.SemaphoreType.DMA,
            pltpu.SemaphoreType.DMA((2,)),
            pltpu.SemaphoreType.DMA((2, N_SUB)),
            pltpu.SemaphoreType.DMA((N_CHUNKS,)),
            pltpu.SemaphoreType.DMA((2,)),
            pltpu.SemaphoreType.DMA((N_CHUNKS,)),
        ],
        compiler_params=pltpu.CompilerParams(
            collective_id=0,
            vmem_limit_bytes=56 * 1024 * 1024,
        ),
    )(O, Wo)
    return out


# device time: 263565 ns/iter; 4.5970x vs baseline; 1.0266x over previous
import jax
import jax.numpy as jnp
from jax import lax
from jax.experimental import pallas as pl
from jax.experimental.pallas import tpu as pltpu

B, S, HD_SHARD, N = 4, 1024, 2048, 4096
H, D = 16, 128
S_HALF = S // 2
M_TILE = 512
N_TILE = 1024
N_SUB = 2
N_CHUNKS = B * N_SUB


def _a_index_map(g, n):
    my_y = lax.axis_index("y")
    h = jnp.where(g % 2 == 0, 1 - my_y, my_y)
    return (g // 2, h, 0, 0)


def _w_index_map(g, n):
    my_x = lax.axis_index("x")
    return (0, my_x * N_SUB + n)


def _body(a_ref, w_ref, out_ref, send_buf, land_ref, out_slots,
          store_sems, ysend_sems, yrecv_sems, xsend_sems, xrecv_sems):
    g = pl.program_id(0)
    n = pl.program_id(1)
    my_x = lax.axis_index("x")
    my_y = lax.axis_index("y")
    ypeer = (my_x, 1 - my_y)
    xpeer = (1 - my_x, my_y)
    col0 = my_x * (N_SUB * N_TILE)

    def y_rdma(tile, cn):
        return pltpu.make_async_remote_copy(
            src_ref=send_buf.at[tile % 2, :, pl.ds(cn * N_TILE, N_TILE)],
            dst_ref=land_ref.at[tile * N_SUB + cn],
            send_sem=ysend_sems.at[tile % 2, cn],
            recv_sem=yrecv_sems.at[tile * N_SUB + cn],
            device_id=ypeer,
            device_id_type=pl.DeviceIdType.MESH,
        )

    def x_rdma(c, tile, cn):
        return pltpu.make_async_remote_copy(
            src_ref=out_slots.at[c % 2],
            dst_ref=out_ref.at[tile, :, pl.ds(col0 + cn * N_TILE, N_TILE)],
            send_sem=xsend_sems.at[c % 2],
            recv_sem=xrecv_sems.at[c],
            device_id=xpeer,
            device_id_type=pl.DeviceIdType.MESH,
        )

    def out_store(c, tile, cn):
        return pltpu.make_async_copy(
            out_slots.at[c % 2],
            out_ref.at[tile, :, pl.ds(col0 + cn * N_TILE, N_TILE)],
            store_sems.at[c % 2],
        )

    tile = g // 2

    @pl.when((g == 0) & (n == 0))
    def _():
        barrier_sem = pltpu.get_barrier_semaphore()
        for nbr in (ypeer, xpeer):
            pl.semaphore_signal(
                barrier_sem,
                inc=1,
                device_id=nbr,
                device_id_type=pl.DeviceIdType.MESH,
            )
        pl.semaphore_wait(barrier_sem, 2)

    @pl.when((g >= 4) & (g % 2 == 0) & (n == 0))
    def _():
        for cn in range(N_SUB):
            y_rdma(tile - 2, cn).wait_send()

    acc = jnp.dot(
        a_ref[0, :, 0, :], w_ref[0:D, :], preferred_element_type=jnp.float32
    )
    for h in range(1, H):
        acc += jnp.dot(
            a_ref[0, :, h, :],
            w_ref[h * D : (h + 1) * D, :],
            preferred_element_type=jnp.float32,
        )

    @pl.when(g % 2 == 0)
    def _():
        send_buf[tile % 2, :, pl.ds(n * N_TILE, N_TILE)] = acc
        y_rdma(tile, n).start()

    @pl.when(g % 2 == 1)
    def _():
        c = tile * N_SUB + n
        @pl.when(c >= 2)
        def _():
            out_store(c - 2, 0, 0).wait()
            x_rdma(c - 2, 0, 0).wait_send()

        y_rdma(tile, n).wait_recv()
        out_slots[c % 2] = acc + land_ref[c]
        out_store(c, tile, n).start()
        x_rdma(c, tile, n).start()

    @pl.when((g == 7) & (n == N_SUB - 1))
    def _():
        for t in (2, 3):
            for cn in range(N_SUB):
                y_rdma(t, cn).wait_send()
        for c in (N_CHUNKS - 2, N_CHUNKS - 1):
            out_store(c, 0, 0).wait()
            x_rdma(c, 0, 0).wait_send()
        for c in range(N_CHUNKS):
            x_rdma(c, 0, 0).wait_recv()


def kernel(O, Wo):
    out = pl.pallas_call(
        _body,
        grid=(2 * B, N_SUB),
        in_specs=[
            pl.BlockSpec((1, M_TILE, H, D), _a_index_map),
            pl.BlockSpec((HD_SHARD, N_TILE), _w_index_map),
        ],
        out_specs=pl.BlockSpec(memory_space=pl.ANY),
        out_shape=jax.ShapeDtypeStruct((B, S_HALF, N), jnp.float32),
        scratch_shapes=[
            pltpu.VMEM((2, M_TILE, N_SUB * N_TILE), jnp.float32),
            pltpu.VMEM((N_CHUNKS, M_TILE, N_TILE), jnp.float32),
            pltpu.VMEM((2, M_TILE, N_TILE), jnp.float32),
            pltpu.SemaphoreType.DMA((2,)),
            pltpu.SemaphoreType.DMA((2, N_SUB)),
            pltpu.SemaphoreType.DMA((N_CHUNKS,)),
            pltpu.SemaphoreType.DMA((2,)),
            pltpu.SemaphoreType.DMA((N_CHUNKS,)),
        ],
        compiler_params=pltpu.CompilerParams(
            collective_id=0,
            vmem_limit_bytes=56 * 1024 * 1024,
        ),
    )(O, Wo)
    return out
